# baseline (speedup 1.0000x reference)
import functools

import jax
import jax.numpy as jnp
from jax import lax
from jax.experimental import pallas as pl
from jax.experimental.pallas import tpu as pltpu
from jax.experimental.pallas import tpu_sc as plsc

TOP_K = 8


def _matmul_kernel(x_ref, w_ref, out_ref):
    out_ref[...] = jax.lax.dot_general(
        x_ref[...], w_ref[...], dimension_numbers=(((1,), (1,)), ((), ())),
        preferred_element_type=jnp.float32)


def _tc_logits(x, W):
    n, d = x.shape
    num_experts = W.shape[0]
    block_t = 1024
    nblocks = n // block_t
    return pl.pallas_call(
        _matmul_kernel,
        grid=(nblocks,),
        in_specs=[
            pl.BlockSpec((block_t, d), lambda i: (i, 0)),
            pl.BlockSpec((num_experts, d), lambda i: (0, 0)),
        ],
        out_specs=pl.BlockSpec((block_t, num_experts), lambda i: (i, 0)),
        out_shape=jax.ShapeDtypeStruct((n, num_experts), jnp.float32),
    )(x, W)


def _make_sc_epilogue(n, num_experts):
    info = plsc.get_sparse_core_info()
    nc, ns, lanes = info.num_cores, info.num_subcores, info.num_lanes
    nw = nc * ns
    chunk = n // nw
    ngroups = chunk // lanes
    acc_len = num_experts * lanes
    mesh = plsc.VectorSubcoreMesh(core_axis_name="c", subcore_axis_name="s")

    @functools.partial(
        pl.kernel, mesh=mesh,
        compiler_params=pltpu.CompilerParams(needs_layout_passes=False),
        out_type=[
            jax.ShapeDtypeStruct((n * TOP_K,), jnp.float32),
            jax.ShapeDtypeStruct((n * TOP_K,), jnp.int32),
            jax.ShapeDtypeStruct((nw, acc_len), jnp.float32),
            jax.ShapeDtypeStruct((nw, acc_len), jnp.float32),
        ],
        scratch_types=[
            pltpu.VMEM((chunk * num_experts,), jnp.float32),
            pltpu.VMEM((chunk * TOP_K,), jnp.float32),
            pltpu.VMEM((chunk * TOP_K,), jnp.int32),
            pltpu.VMEM((acc_len,), jnp.float32),
            pltpu.VMEM((acc_len,), jnp.float32),
        ],
    )
    def sc_epilogue(logits_hbm, wout_hbm, iout_hbm, pout_hbm, fout_hbm,
                    lg_v, wv, iv, pacc, facc):
        wid = lax.axis_index("s") * nc + lax.axis_index("c")
        base = wid * chunk
        pltpu.sync_copy(logits_hbm.at[pl.ds(base * num_experts,
                                            chunk * num_experts)], lg_v)

        iota = jnp.arange(lanes, dtype=jnp.int32)
        zeros = jnp.zeros((lanes,), jnp.float32)
        ones = jnp.ones((lanes,), jnp.float32)
        for e in range(num_experts):
            pacc[pl.ds(e * lanes, lanes)] = zeros
            facc[pl.ds(e * lanes, lanes)] = zeros

        def group_body(g, carry):
            gbase = g * lanes * num_experts + iota * num_experts
            lg = [plsc.load_gather(lg_v, [gbase + e])
                  for e in range(num_experts)]
            rmax = functools.reduce(jnp.maximum, lg)
            ee = [jnp.exp(v - rmax) for v in lg]
            denom = functools.reduce(jnp.add, ee)
            rden = 1.0 / denom
            for e in range(num_experts):
                plsc.addupdate(pacc.at[pl.ds(e * lanes, lanes)],
                               ee[e] * rden)

            mask_lo = num_experts - 1
            keys = [
                lax.bitcast_convert_type(
                    (lax.bitcast_convert_type(ee[e], jnp.int32) & ~mask_lo)
                    | (mask_lo - e), jnp.float32)
                for e in range(num_experts)
            ]
            rowk = g * lanes * TOP_K + iota * TOP_K
            vals = []
            for j in range(TOP_K):
                m = functools.reduce(jnp.maximum, keys)
                keys = [jnp.where(k == m, -1.0, k) for k in keys]
                mbits = lax.bitcast_convert_type(m, jnp.int32)
                idx_j = mask_lo - (mbits & mask_lo)
                val_j = lax.bitcast_convert_type(mbits & ~mask_lo, jnp.float32)
                vals.append(val_j)
                plsc.addupdate_scatter(facc, [idx_j * lanes + iota], ones)
                plsc.store_scatter(iv, [rowk + j], idx_j)
            rsum = 1.0 / functools.reduce(jnp.add, vals)
            for j in range(TOP_K):
                plsc.store_scatter(wv, [rowk + j], vals[j] * rsum)
            return carry

        lax.fori_loop(0, ngroups, group_body, 0)

        pltpu.sync_copy(wv, wout_hbm.at[pl.ds(base * TOP_K, chunk * TOP_K)])
        pltpu.sync_copy(iv, iout_hbm.at[pl.ds(base * TOP_K, chunk * TOP_K)])
        pltpu.sync_copy(pacc, pout_hbm.at[wid])
        pltpu.sync_copy(facc, fout_hbm.at[wid])

    return sc_epilogue


@jax.jit
def kernel(x, W):
    n, _ = x.shape
    num_experts = W.shape[0]
    logits = _tc_logits(x, W)
    wflat, iflat, pparts, fparts = _make_sc_epilogue(n, num_experts)(
        logits.reshape(-1))
    p_i = jnp.sum(pparts.reshape(-1, num_experts, 16), axis=(0, 2)) / n
    f_i = jnp.sum(fparts.reshape(-1, num_experts, 16), axis=(0, 2)) / n
    aux = num_experts * jnp.sum(f_i * p_i)
    return wflat.reshape(n, TOP_K), iflat.reshape(n, TOP_K), aux


# speedup vs baseline: 1.3316x; 1.3316x over previous
import functools

import jax
import jax.numpy as jnp
from jax.experimental import pallas as pl
from jax.experimental.pallas import tpu as pltpu

TOP_K = 8


def _router_kernel(x_ref, w_ref, wout_ref, iout_ref, aux_ref, lacc_ref,
                   acc_ref, *, nblocks, nk, dk, n_tokens, num_experts):
    i = pl.program_id(0)
    j = pl.program_id(1)
    xb = x_ref[...]
    wt = w_ref[:, pl.ds(j * dk, dk)]
    partial = jax.lax.dot_general(
        xb, wt, dimension_numbers=(((1,), (1,)), ((), ())),
        preferred_element_type=jnp.float32)

    @pl.when(j == 0)
    def _first():
        lacc_ref[...] = partial

    @pl.when(j == nk - 1)
    def _epilogue():
        logits = lacc_ref[...] + partial

        row_max = jnp.max(logits, axis=-1, keepdims=True)
        e = jnp.exp(logits - row_max)
        denom = jnp.sum(e, axis=-1, keepdims=True)
        p_part = jnp.sum(e * (1.0 / denom), axis=0)

        t = logits.shape[0]
        iota = jax.lax.broadcasted_iota(jnp.int32, (t, num_experts), 1)
        ebits = jax.lax.bitcast_convert_type(e, jnp.int32)
        key = jax.lax.bitcast_convert_type(
            (ebits & ~(num_experts - 1)) | (num_experts - 1 - iota),
            jnp.float32)

        sel_mask = jnp.zeros((t, num_experts), jnp.float32)
        ms = []
        for _ in range(TOP_K):
            m = jnp.max(key, axis=-1, keepdims=True)
            hit = key == m
            sel_mask = sel_mask + hit.astype(jnp.float32)
            key = jnp.where(hit, -1.0, key)
            ms.append(m)

        mcat = jax.lax.bitcast_convert_type(
            jnp.concatenate(ms, axis=-1), jnp.int32)
        w_top = jax.lax.bitcast_convert_type(
            mcat & ~(num_experts - 1), jnp.float32)
        wout_ref[...] = w_top / jnp.sum(w_top, axis=-1, keepdims=True)
        iout_ref[...] = (num_experts - 1) - (mcat & (num_experts - 1))

        f_part = jnp.sum(sel_mask, axis=0)

        @pl.when(i == 0)
        def _init():
            acc_ref[...] = jnp.zeros_like(acc_ref)

        acc_ref[0:1, :] += p_part[None, :]
        acc_ref[1:2, :] += f_part[None, :]

        @pl.when(i == nblocks - 1)
        def _finish():
            scale = num_experts / (float(n_tokens) * float(n_tokens))
            aux = scale * jnp.sum(acc_ref[0:1, :] * acc_ref[1:2, :],
                                  axis=-1, keepdims=True)
            aux_ref[...] = aux


@jax.jit
def kernel(x, W):
    n, d = x.shape
    num_experts = W.shape[0]
    block_t = 1024
    nblocks = n // block_t
    nk = 2
    dk = d // nk

    kern = functools.partial(_router_kernel, nblocks=nblocks, nk=nk, dk=dk,
                             n_tokens=n, num_experts=num_experts)
    weights, indices, aux = pl.pallas_call(
        kern,
        grid=(nblocks, nk),
        in_specs=[
            pl.BlockSpec((block_t, dk), lambda i, j: (i, j)),
            pl.BlockSpec((num_experts, d), lambda i, j: (0, 0)),
        ],
        out_specs=[
            pl.BlockSpec((block_t, TOP_K), lambda i, j: (i, 0)),
            pl.BlockSpec((block_t, TOP_K), lambda i, j: (i, 0)),
            pl.BlockSpec((1, 1), lambda i, j: (0, 0)),
        ],
        out_shape=[
            jax.ShapeDtypeStruct((n, TOP_K), jnp.float32),
            jax.ShapeDtypeStruct((n, TOP_K), jnp.int32),
            jax.ShapeDtypeStruct((1, 1), jnp.float32),
        ],
        scratch_shapes=[
            pltpu.VMEM((block_t, num_experts), jnp.float32),
            pltpu.VMEM((2, num_experts), jnp.float32),
        ],
    )(x, W)
    return weights, indices, aux[0, 0]


# speedup vs baseline: 1.6594x; 1.2462x over previous
import functools

import jax
import jax.numpy as jnp
from jax.experimental import pallas as pl
from jax.experimental.pallas import tpu as pltpu

TOP_K = 8


def _router_kernel(x_ref, w_ref, wout_ref, iout_ref, aux_ref, acc_ref,
                   *, nblocks, n_tokens, num_experts):
    i = pl.program_id(0)
    xb = x_ref[...]
    wt = w_ref[...]
    logits = jax.lax.dot_general(
        xb, wt, dimension_numbers=(((1,), (1,)), ((), ())),
        preferred_element_type=jnp.float32)

    row_max = jnp.max(logits, axis=-1, keepdims=True)
    e = jnp.exp(logits - row_max)
    denom = jnp.sum(e, axis=-1, keepdims=True)
    p_part = jnp.sum(e * (1.0 / denom), axis=0)

    t = logits.shape[0]
    iota = jax.lax.broadcasted_iota(jnp.int32, (t, num_experts), 1)
    ebits = jax.lax.bitcast_convert_type(e, jnp.int32) + (num_experts // 2)
    key = jax.lax.bitcast_convert_type(
        (ebits & ~(num_experts - 1)) | (num_experts - 1 - iota), jnp.float32)

    sel_mask = jnp.zeros((t, num_experts), jnp.float32)
    ms = []
    for _ in range(TOP_K):
        m = jnp.max(key, axis=-1, keepdims=True)
        hit = key == m
        sel_mask = sel_mask + hit.astype(jnp.float32)
        key = jnp.where(hit, -1.0, key)
        ms.append(m)

    mcat = jax.lax.bitcast_convert_type(
        jnp.concatenate(ms, axis=-1), jnp.int32)
    w_top = jax.lax.bitcast_convert_type(
        mcat & ~(num_experts - 1), jnp.float32)
    wout_ref[...] = w_top / jnp.sum(w_top, axis=-1, keepdims=True)
    iout_ref[...] = (num_experts - 1) - (mcat & (num_experts - 1))

    f_part = jnp.sum(sel_mask, axis=0)

    @pl.when(i == 0)
    def _init():
        acc_ref[...] = jnp.zeros_like(acc_ref)

    acc_ref[0:1, :] += p_part[None, :]
    acc_ref[1:2, :] += f_part[None, :]

    @pl.when(i == nblocks - 1)
    def _finish():
        scale = num_experts / (float(n_tokens) * float(n_tokens))
        aux = scale * jnp.sum(acc_ref[0:1, :] * acc_ref[1:2, :],
                              axis=-1, keepdims=True)
        aux_ref[...] = aux


@jax.jit
def kernel(x, W):
    n, d = x.shape
    num_experts = W.shape[0]
    block_t = 1024 if n % 1024 == 0 else n
    nblocks = n // block_t

    kern = functools.partial(_router_kernel, nblocks=nblocks, n_tokens=n,
                             num_experts=num_experts)
    weights, indices, aux = pl.pallas_call(
        kern,
        grid=(nblocks,),
        in_specs=[
            pl.BlockSpec((block_t, d), lambda i: (i, 0)),
            pl.BlockSpec((num_experts, d), lambda i: (0, 0)),
        ],
        out_specs=[
            pl.BlockSpec((block_t, TOP_K), lambda i: (i, 0)),
            pl.BlockSpec((block_t, TOP_K), lambda i: (i, 0)),
            pl.BlockSpec((1, 1), lambda i: (0, 0)),
        ],
        out_shape=[
            jax.ShapeDtypeStruct((n, TOP_K), jnp.float32),
            jax.ShapeDtypeStruct((n, TOP_K), jnp.int32),
            jax.ShapeDtypeStruct((1, 1), jnp.float32),
        ],
        scratch_shapes=[pltpu.VMEM((2, num_experts), jnp.float32)],
    )(x, W)
    return weights, indices, aux[0, 0]
